# Initial kernel scaffold; baseline (speedup 1.0000x reference)
import jax
import jax.numpy as jnp
from jax.experimental import pallas as pl


def kernel(x, edge_index, W1, b1, W2, b2):
    raise NotImplementedError("write your pallas kernel here")


# speedup vs baseline: 117.0447x; 117.0447x over previous
import functools

import jax
import jax.numpy as jnp
from jax import lax
from jax.experimental import pallas as pl
from jax.experimental.pallas import tpu as pltpu
from jax.experimental.pallas import tpu_sc as plsc

N_NODES = 100000
NP = 100352
NPR = 784
NC, NS = 2, 16
NW = NC * NS
SLICE = NP // NS
ROWL = 128
CROWS = 64


def _worker_id():
    return lax.axis_index("s") * NC + lax.axis_index("c")


def _fill(ref, n, value):
    v = jnp.full((16,), value, jnp.float32)

    def body(i, _):
        ref[pl.ds(i * 16, 16)] = v
        return 0

    lax.fori_loop(0, n // 16, body, 0)


def _edge_loop(nchunks, rows_per_w, body):
    w = _worker_id()

    def chunk(ch, _):
        body(w * rows_per_w + ch * CROWS)
        return 0

    lax.fori_loop(0, nchunks, chunk, 0)


@functools.lru_cache(maxsize=None)
def _sc_degree(nrows):
    rpw = nrows // NW
    nch = rpw // CROWS
    mesh = plsc.VectorSubcoreMesh(core_axis_name="c", subcore_axis_name="s")

    def body(dst_hbm, out_hbm, acc_sp, dstv, ones_v, zbuf, sem_s):
        c = lax.axis_index("c")
        s = lax.axis_index("s")
        _fill(ones_v, ROWL, 1.0)
        _fill(zbuf, SLICE, 0.0)
        base = s * SLICE
        pltpu.sync_copy(zbuf, acc_sp.at[pl.ds(base, SLICE)])
        plsc.subcore_barrier()

        def work(row0):
            pltpu.sync_copy(dst_hbm.at[pl.ds(row0, CROWS)], dstv)
            descs = [
                pltpu.async_copy(ones_v, acc_sp.at[dstv.at[j]], sem_s, add=True)
                for j in range(CROWS)
            ]
            for d in descs:
                d.wait()

        _edge_loop(nch, rpw, work)
        plsc.subcore_barrier()
        pltpu.sync_copy(acc_sp.at[pl.ds(base, SLICE)],
                        out_hbm.at[c, pl.ds(base, SLICE)])

    return pl.kernel(
        body,
        out_type=jax.ShapeDtypeStruct((NC, NP), jnp.float32),
        mesh=mesh,
        scratch_types=[
            pltpu.VMEM_SHARED((NP,), jnp.float32),
            pltpu.VMEM((CROWS, ROWL), jnp.int32),
            pltpu.VMEM((ROWL,), jnp.float32),
            pltpu.VMEM((SLICE,), jnp.float32),
            pltpu.SemaphoreType.DMA,
        ],
    )


@functools.lru_cache(maxsize=None)
def _sc_segsum1(nrows):
    rpw = nrows // NW
    nch = rpw // CROWS
    mesh = plsc.VectorSubcoreMesh(core_axis_name="c", subcore_axis_name="s")

    def body(src_hbm, dst_hbm, g_hbm, out_hbm,
             g_sp, acc_sp, srcv, dstv, vals, zbuf, sem_g, sem_s):
        c = lax.axis_index("c")
        s = lax.axis_index("s")
        _fill(zbuf, SLICE, 0.0)
        base = s * SLICE
        pltpu.sync_copy(zbuf, acc_sp.at[pl.ds(base, SLICE)])
        pltpu.sync_copy(g_hbm.at[pl.ds(base, SLICE)],
                        g_sp.at[pl.ds(base, SLICE)])
        plsc.subcore_barrier()

        def work(row0):
            pltpu.sync_copy(src_hbm.at[pl.ds(row0, CROWS)], srcv)
            pltpu.sync_copy(dst_hbm.at[pl.ds(row0, CROWS)], dstv)
            gd = [
                pltpu.async_copy(g_sp.at[srcv.at[j]],
                                 vals.at[pl.ds(j * ROWL, ROWL)], sem_g)
                for j in range(CROWS)
            ]
            for d in gd:
                d.wait()
            sd = [
                pltpu.async_copy(vals.at[pl.ds(j * ROWL, ROWL)],
                                 acc_sp.at[dstv.at[j]], sem_s, add=True)
                for j in range(CROWS)
            ]
            for d in sd:
                d.wait()

        _edge_loop(nch, rpw, work)
        plsc.subcore_barrier()
        pltpu.sync_copy(acc_sp.at[pl.ds(base, SLICE)],
                        out_hbm.at[c, pl.ds(base, SLICE)])

    return pl.kernel(
        body,
        out_type=jax.ShapeDtypeStruct((NC, NP), jnp.float32),
        mesh=mesh,
        scratch_types=[
            pltpu.VMEM_SHARED((NP,), jnp.float32),
            pltpu.VMEM_SHARED((NP,), jnp.float32),
            pltpu.VMEM((CROWS, ROWL), jnp.int32),
            pltpu.VMEM((CROWS, ROWL), jnp.int32),
            pltpu.VMEM((CROWS * ROWL,), jnp.float32),
            pltpu.VMEM((SLICE,), jnp.float32),
            pltpu.SemaphoreType.DMA,
            pltpu.SemaphoreType.DMA,
        ],
    )


@functools.lru_cache(maxsize=None)
def _sc_segsum2(nrows):
    rpw = nrows // NW
    nch = rpw // CROWS
    mesh = plsc.VectorSubcoreMesh(core_axis_name="c", subcore_axis_name="s")

    def body(src_hbm, dst_hbm, ga_hbm, gb_hbm, oa_hbm, ob_hbm,
             ga_sp, gb_sp, aa_sp, ab_sp, srcv, dstv, va, vb, zbuf,
             sem_g, sem_s):
        c = lax.axis_index("c")
        s = lax.axis_index("s")
        _fill(zbuf, SLICE, 0.0)
        base = s * SLICE
        pltpu.sync_copy(zbuf, aa_sp.at[pl.ds(base, SLICE)])
        pltpu.sync_copy(zbuf, ab_sp.at[pl.ds(base, SLICE)])
        pltpu.sync_copy(ga_hbm.at[pl.ds(base, SLICE)],
                        ga_sp.at[pl.ds(base, SLICE)])
        pltpu.sync_copy(gb_hbm.at[pl.ds(base, SLICE)],
                        gb_sp.at[pl.ds(base, SLICE)])
        plsc.subcore_barrier()

        def work(row0):
            pltpu.sync_copy(src_hbm.at[pl.ds(row0, CROWS)], srcv)
            pltpu.sync_copy(dst_hbm.at[pl.ds(row0, CROWS)], dstv)
            gd = [
                pltpu.async_copy(ga_sp.at[srcv.at[j]],
                                 va.at[pl.ds(j * ROWL, ROWL)], sem_g)
                for j in range(CROWS)
            ] + [
                pltpu.async_copy(gb_sp.at[srcv.at[j]],
                                 vb.at[pl.ds(j * ROWL, ROWL)], sem_g)
                for j in range(CROWS)
            ]
            for d in gd:
                d.wait()
            sd = [
                pltpu.async_copy(va.at[pl.ds(j * ROWL, ROWL)],
                                 aa_sp.at[dstv.at[j]], sem_s, add=True)
                for j in range(CROWS)
            ] + [
                pltpu.async_copy(vb.at[pl.ds(j * ROWL, ROWL)],
                                 ab_sp.at[dstv.at[j]], sem_s, add=True)
                for j in range(CROWS)
            ]
            for d in sd:
                d.wait()

        _edge_loop(nch, rpw, work)
        plsc.subcore_barrier()
        pltpu.sync_copy(aa_sp.at[pl.ds(base, SLICE)],
                        oa_hbm.at[c, pl.ds(base, SLICE)])
        pltpu.sync_copy(ab_sp.at[pl.ds(base, SLICE)],
                        ob_hbm.at[c, pl.ds(base, SLICE)])

    return pl.kernel(
        body,
        out_type=(jax.ShapeDtypeStruct((NC, NP), jnp.float32),
                  jax.ShapeDtypeStruct((NC, NP), jnp.float32)),
        mesh=mesh,
        scratch_types=[
            pltpu.VMEM_SHARED((NP,), jnp.float32),
            pltpu.VMEM_SHARED((NP,), jnp.float32),
            pltpu.VMEM_SHARED((NP,), jnp.float32),
            pltpu.VMEM_SHARED((NP,), jnp.float32),
            pltpu.VMEM((CROWS, ROWL), jnp.int32),
            pltpu.VMEM((CROWS, ROWL), jnp.int32),
            pltpu.VMEM((CROWS * ROWL,), jnp.float32),
            pltpu.VMEM((CROWS * ROWL,), jnp.float32),
            pltpu.VMEM((SLICE,), jnp.float32),
            pltpu.SemaphoreType.DMA,
            pltpu.SemaphoreType.DMA,
        ],
    )


def _tc_prep_body(degp_ref, x_ref, dinv_ref, g1_ref):
    deg = degp_ref[0] + degp_ref[1] + 1.0
    dinv = lax.rsqrt(deg)
    dinv_ref[...] = dinv
    g1_ref[...] = dinv * x_ref[...]


_tc_prep = pl.pallas_call(
    _tc_prep_body,
    out_shape=(jax.ShapeDtypeStruct((NPR, 128), jnp.float32),
               jax.ShapeDtypeStruct((NPR, 128), jnp.float32)),
)


def _tc_mid_body(s_ref, dinv_ref, x_ref, w1_ref, b1_ref, w2_ref,
                 ha_ref, hb_ref, ga_ref, gb_ref):
    dinv = dinv_ref[...]
    t = dinv * (s_ref[0] + s_ref[1]) + dinv * dinv * x_ref[...]
    acc0 = jnp.zeros_like(t)
    acc1 = jnp.zeros_like(t)
    for k in range(16):
        h = jnp.maximum(t * w1_ref[0, k] + b1_ref[0, k], 0.0)
        acc0 = acc0 + h * w2_ref[k, 0]
        acc1 = acc1 + h * w2_ref[k, 1]
    ha_ref[...] = acc0
    hb_ref[...] = acc1
    ga_ref[...] = dinv * acc0
    gb_ref[...] = dinv * acc1


_tc_mid = pl.pallas_call(
    _tc_mid_body,
    out_shape=tuple(jax.ShapeDtypeStruct((NPR, 128), jnp.float32)
                    for _ in range(4)),
)


def _tc_final_body(oa_ref, ob_ref, dinv_ref, ha_ref, hb_ref, b2_ref,
                   outa_ref, outb_ref):
    dinv = dinv_ref[...]
    d2 = dinv * dinv
    outa_ref[...] = dinv * (oa_ref[0] + oa_ref[1]) + d2 * ha_ref[...] + b2_ref[0, 0]
    outb_ref[...] = dinv * (ob_ref[0] + ob_ref[1]) + d2 * hb_ref[...] + b2_ref[0, 1]


_tc_final = pl.pallas_call(
    _tc_final_body,
    out_shape=(jax.ShapeDtypeStruct((NPR, 128), jnp.float32),
               jax.ShapeDtypeStruct((NPR, 128), jnp.float32)),
)


def kernel(x, edge_index, W1, b1, W2, b2):
    n = x.shape[0]
    e = edge_index.shape[1]
    chunk_edges = NW * CROWS * ROWL
    ep = ((e + chunk_edges - 1) // chunk_edges) * chunk_edges
    nrows = ep // ROWL

    pad_idx = jnp.full((ep - e,), NP - 1, edge_index.dtype)
    srcp = jnp.concatenate([edge_index[0], pad_idx]).reshape(nrows, ROWL)
    dstp = jnp.concatenate([edge_index[1], pad_idx]).reshape(nrows, ROWL)
    xf = jnp.concatenate(
        [x[:, 0], jnp.zeros((NP - n,), x.dtype)]).reshape(NPR, 128)

    deg2 = _sc_degree(nrows)(dstp).reshape(NC, NPR, 128)
    dinv, g1 = _tc_prep(deg2, xf)
    s2 = _sc_segsum1(nrows)(srcp, dstp, g1.reshape(NP)).reshape(NC, NPR, 128)
    ha, hb, ga, gb = _tc_mid(s2, dinv, xf, W1, b1.reshape(1, 16), W2)
    oa, ob = _sc_segsum2(nrows)(srcp, dstp, ga.reshape(NP), gb.reshape(NP))
    outa, outb = _tc_final(oa.reshape(NC, NPR, 128), ob.reshape(NC, NPR, 128),
                           dinv, ha, hb, b2.reshape(1, 2))
    return jnp.stack([outa.reshape(NP)[:n], outb.reshape(NP)[:n]], axis=1)


# speedup vs baseline: 135.2756x; 1.1558x over previous
import functools

import jax
import jax.numpy as jnp
from jax import lax
from jax.experimental import pallas as pl
from jax.experimental.pallas import tpu as pltpu
from jax.experimental.pallas import tpu_sc as plsc

N_NODES = 100000
NP = 100352
NPR = 784
NC, NS = 2, 16
NW = NC * NS
SLICE = NP // NS
ROWL = 128
CROWS = 64


def _worker_id():
    return lax.axis_index("s") * NC + lax.axis_index("c")


def _fill(ref, n, value):
    v = jnp.full((16,), value, jnp.float32)

    def body(i, _):
        ref[pl.ds(i * 16, 16)] = v
        return 0

    lax.fori_loop(0, n // 16, body, 0)


def _edge_loop(nchunks, rows_per_w, body):
    w = _worker_id()

    def chunk(ch, _):
        body(w * rows_per_w + ch * CROWS)
        return 0

    lax.fori_loop(0, nchunks, chunk, 0)


@functools.lru_cache(maxsize=None)
def _sc_degree(nrows):
    rpw = nrows // NW
    nch = rpw // CROWS
    mesh = plsc.VectorSubcoreMesh(core_axis_name="c", subcore_axis_name="s")

    def body(dst_hbm, out_hbm, acc_sp, dstv, ones_v, zbuf, sem_s):
        c = lax.axis_index("c")
        s = lax.axis_index("s")
        _fill(ones_v, ROWL, 1.0)
        _fill(zbuf, SLICE, 0.0)
        base = s * SLICE
        pltpu.sync_copy(zbuf, acc_sp.at[pl.ds(base, SLICE)])
        plsc.subcore_barrier()

        def work(row0):
            pltpu.sync_copy(dst_hbm.at[pl.ds(row0, CROWS)], dstv)
            descs = [
                pltpu.async_copy(ones_v, acc_sp.at[dstv.at[j]], sem_s, add=True)
                for j in range(CROWS)
            ]
            for d in descs:
                d.wait()

        _edge_loop(nch, rpw, work)
        plsc.subcore_barrier()
        pltpu.sync_copy(acc_sp.at[pl.ds(base, SLICE)],
                        out_hbm.at[c, pl.ds(base, SLICE)])

    return pl.kernel(
        body,
        out_type=jax.ShapeDtypeStruct((NC, NP), jnp.float32),
        mesh=mesh,
        scratch_types=[
            pltpu.VMEM_SHARED((NP,), jnp.float32),
            pltpu.VMEM((CROWS, ROWL), jnp.int32),
            pltpu.VMEM((ROWL,), jnp.float32),
            pltpu.VMEM((SLICE,), jnp.float32),
            pltpu.SemaphoreType.DMA,
        ],
    )


CROWS2 = 32


def _gather_rows(tab, srcv, vals):
    def body(i, _):
        j = i // (ROWL // 16)
        k = i % (ROWL // 16)
        idx = srcv[j, pl.ds(k * 16, 16)]
        vals[j, pl.ds(k * 16, 16)] = plsc.load_gather(tab, [idx])
        return 0

    lax.fori_loop(0, CROWS2 * (ROWL // 16), body, 0)


def _zero_acc(acc_sp, zbuf, base):
    _fill(zbuf, SLICE, 0.0)
    pltpu.sync_copy(zbuf, acc_sp.at[pl.ds(base, SLICE)])


@functools.lru_cache(maxsize=None)
def _sc_segsum1(nrows):
    rpw = nrows // NW
    nch = rpw // CROWS2
    mesh = plsc.VectorSubcoreMesh(core_axis_name="c", subcore_axis_name="s")

    def body(src_hbm, dst_hbm, g_hbm, out_hbm,
             acc_sp, tab, srcv, dstv, vals, zbuf, sem_s):
        c = lax.axis_index("c")
        s = lax.axis_index("s")
        base = s * SLICE
        _zero_acc(acc_sp, zbuf, base)
        pltpu.sync_copy(g_hbm, tab)
        plsc.subcore_barrier()
        w = s * NC + c

        def chunk(ch, _):
            row0 = w * rpw + ch * CROWS2
            pltpu.sync_copy(src_hbm.at[pl.ds(row0, CROWS2)], srcv)
            pltpu.sync_copy(dst_hbm.at[pl.ds(row0, CROWS2)], dstv)
            _gather_rows(tab, srcv, vals)
            sd = [
                pltpu.async_copy(vals.at[j], acc_sp.at[dstv.at[j]],
                                 sem_s, add=True)
                for j in range(CROWS2)
            ]
            for d in sd:
                d.wait()
            return 0

        lax.fori_loop(0, nch, chunk, 0)
        plsc.subcore_barrier()
        pltpu.sync_copy(acc_sp.at[pl.ds(base, SLICE)],
                        out_hbm.at[c, pl.ds(base, SLICE)])

    return pl.kernel(
        body,
        out_type=jax.ShapeDtypeStruct((NC, NP), jnp.float32),
        mesh=mesh,
        compiler_params=pltpu.CompilerParams(needs_layout_passes=False),
        scratch_types=[
            pltpu.VMEM_SHARED((NP,), jnp.float32),
            pltpu.VMEM((NP,), jnp.float32),
            pltpu.VMEM((CROWS2, ROWL), jnp.int32),
            pltpu.VMEM((CROWS2, ROWL), jnp.int32),
            pltpu.VMEM((CROWS2, ROWL), jnp.float32),
            pltpu.VMEM((SLICE,), jnp.float32),
            pltpu.SemaphoreType.DMA,
        ],
    )


@functools.lru_cache(maxsize=None)
def _sc_segsum2(nrows):
    rpw = nrows // NS
    nch = rpw // CROWS2
    mesh = plsc.VectorSubcoreMesh(core_axis_name="c", subcore_axis_name="s")

    def body(src_hbm, dst_hbm, ga_hbm, gb_hbm, out_hbm,
             acc_sp, tab, srcv, dstv, vals, zbuf, sem_s):
        c = lax.axis_index("c")
        s = lax.axis_index("s")
        base = s * SLICE
        _zero_acc(acc_sp, zbuf, base)

        @pl.when(c == 0)
        def _():
            pltpu.sync_copy(ga_hbm, tab)

        @pl.when(c == 1)
        def _():
            pltpu.sync_copy(gb_hbm, tab)

        plsc.subcore_barrier()

        def chunk(ch, _):
            row0 = s * rpw + ch * CROWS2
            pltpu.sync_copy(src_hbm.at[pl.ds(row0, CROWS2)], srcv)
            pltpu.sync_copy(dst_hbm.at[pl.ds(row0, CROWS2)], dstv)
            _gather_rows(tab, srcv, vals)
            sd = [
                pltpu.async_copy(vals.at[j], acc_sp.at[dstv.at[j]],
                                 sem_s, add=True)
                for j in range(CROWS2)
            ]
            for d in sd:
                d.wait()
            return 0

        lax.fori_loop(0, nch, chunk, 0)
        plsc.subcore_barrier()
        pltpu.sync_copy(acc_sp.at[pl.ds(base, SLICE)],
                        out_hbm.at[c, pl.ds(base, SLICE)])

    return pl.kernel(
        body,
        out_type=jax.ShapeDtypeStruct((NC, NP), jnp.float32),
        mesh=mesh,
        compiler_params=pltpu.CompilerParams(needs_layout_passes=False),
        scratch_types=[
            pltpu.VMEM_SHARED((NP,), jnp.float32),
            pltpu.VMEM((NP,), jnp.float32),
            pltpu.VMEM((CROWS2, ROWL), jnp.int32),
            pltpu.VMEM((CROWS2, ROWL), jnp.int32),
            pltpu.VMEM((CROWS2, ROWL), jnp.float32),
            pltpu.VMEM((SLICE,), jnp.float32),
            pltpu.SemaphoreType.DMA,
        ],
    )


def _tc_prep_body(degp_ref, x_ref, dinv_ref, g1_ref):
    deg = degp_ref[0] + degp_ref[1] + 1.0
    dinv = lax.rsqrt(deg)
    dinv_ref[...] = dinv
    g1_ref[...] = dinv * x_ref[...]


_tc_prep = pl.pallas_call(
    _tc_prep_body,
    out_shape=(jax.ShapeDtypeStruct((NPR, 128), jnp.float32),
               jax.ShapeDtypeStruct((NPR, 128), jnp.float32)),
)


def _tc_mid_body(s_ref, dinv_ref, x_ref, w1_ref, b1_ref, w2_ref,
                 ha_ref, hb_ref, ga_ref, gb_ref):
    dinv = dinv_ref[...]
    t = dinv * (s_ref[0] + s_ref[1]) + dinv * dinv * x_ref[...]
    acc0 = jnp.zeros_like(t)
    acc1 = jnp.zeros_like(t)
    for k in range(16):
        h = jnp.maximum(t * w1_ref[0, k] + b1_ref[0, k], 0.0)
        acc0 = acc0 + h * w2_ref[k, 0]
        acc1 = acc1 + h * w2_ref[k, 1]
    ha_ref[...] = acc0
    hb_ref[...] = acc1
    ga_ref[...] = dinv * acc0
    gb_ref[...] = dinv * acc1


_tc_mid = pl.pallas_call(
    _tc_mid_body,
    out_shape=tuple(jax.ShapeDtypeStruct((NPR, 128), jnp.float32)
                    for _ in range(4)),
)


def _tc_final_body(o_ref, dinv_ref, ha_ref, hb_ref, b2_ref,
                   outa_ref, outb_ref):
    dinv = dinv_ref[...]
    d2 = dinv * dinv
    outa_ref[...] = dinv * o_ref[0] + d2 * ha_ref[...] + b2_ref[0, 0]
    outb_ref[...] = dinv * o_ref[1] + d2 * hb_ref[...] + b2_ref[0, 1]


_tc_final = pl.pallas_call(
    _tc_final_body,
    out_shape=(jax.ShapeDtypeStruct((NPR, 128), jnp.float32),
               jax.ShapeDtypeStruct((NPR, 128), jnp.float32)),
)


def kernel(x, edge_index, W1, b1, W2, b2):
    n = x.shape[0]
    e = edge_index.shape[1]
    chunk_edges = NW * CROWS * ROWL
    ep = ((e + chunk_edges - 1) // chunk_edges) * chunk_edges
    nrows = ep // ROWL

    pad_idx = jnp.full((ep - e,), NP - 1, edge_index.dtype)
    srcp = jnp.concatenate([edge_index[0], pad_idx]).reshape(nrows, ROWL)
    dstp = jnp.concatenate([edge_index[1], pad_idx]).reshape(nrows, ROWL)
    xf = jnp.concatenate(
        [x[:, 0], jnp.zeros((NP - n,), x.dtype)]).reshape(NPR, 128)

    deg2 = _sc_degree(nrows)(dstp).reshape(NC, NPR, 128)
    dinv, g1 = _tc_prep(deg2, xf)
    s2 = _sc_segsum1(nrows)(srcp, dstp, g1.reshape(NP)).reshape(NC, NPR, 128)
    ha, hb, ga, gb = _tc_mid(s2, dinv, xf, W1, b1.reshape(1, 16), W2)
    o2 = _sc_segsum2(nrows)(srcp, dstp, ga.reshape(NP), gb.reshape(NP))
    outa, outb = _tc_final(o2.reshape(NC, NPR, 128),
                           dinv, ha, hb, b2.reshape(1, 2))
    return jnp.stack([outa.reshape(NP)[:n], outb.reshape(NP)[:n]], axis=1)


# speedup vs baseline: 164.5614x; 1.2165x over previous
import functools

import jax
import jax.numpy as jnp
from jax import lax
from jax.experimental import pallas as pl
from jax.experimental.pallas import tpu as pltpu
from jax.experimental.pallas import tpu_sc as plsc

N_NODES = 100000
NP = 100352
NPR = 784
NC, NS = 2, 16
NW = NC * NS
SLICE = NP // NS
ROWL = 128
CROWS = 64


def _worker_id():
    return lax.axis_index("s") * NC + lax.axis_index("c")


def _fill(ref, n, value):
    v = jnp.full((16,), value, jnp.float32)

    def body(i, _):
        ref[pl.ds(i * 16, 16)] = v
        return 0

    lax.fori_loop(0, n // 16, body, 0)


def _edge_loop(nchunks, rows_per_w, body):
    w = _worker_id()

    def chunk(ch, _):
        body(w * rows_per_w + ch * CROWS)
        return 0

    lax.fori_loop(0, nchunks, chunk, 0)


@functools.lru_cache(maxsize=None)
def _sc_degree(nrows):
    rpw = nrows // NW
    nch = rpw // CROWS
    mesh = plsc.VectorSubcoreMesh(core_axis_name="c", subcore_axis_name="s")

    def body(dst_hbm, out_hbm, acc_sp, dstv, ones_v, zbuf, sem_s):
        c = lax.axis_index("c")
        s = lax.axis_index("s")
        _fill(ones_v, ROWL, 1.0)
        _fill(zbuf, SLICE, 0.0)
        base = s * SLICE
        pltpu.sync_copy(zbuf, acc_sp.at[pl.ds(base, SLICE)])
        plsc.subcore_barrier()

        def work(row0):
            pltpu.sync_copy(dst_hbm.at[pl.ds(row0, CROWS)], dstv)
            descs = [
                pltpu.async_copy(ones_v, acc_sp.at[dstv.at[j]], sem_s, add=True)
                for j in range(CROWS)
            ]
            for d in descs:
                d.wait()

        _edge_loop(nch, rpw, work)
        plsc.subcore_barrier()
        pltpu.sync_copy(acc_sp.at[pl.ds(base, SLICE)],
                        out_hbm.at[c, pl.ds(base, SLICE)])

    return pl.kernel(
        body,
        out_type=jax.ShapeDtypeStruct((NC, NP), jnp.float32),
        mesh=mesh,
        scratch_types=[
            pltpu.VMEM_SHARED((NP,), jnp.float32),
            pltpu.VMEM((CROWS, ROWL), jnp.int32),
            pltpu.VMEM((ROWL,), jnp.float32),
            pltpu.VMEM((SLICE,), jnp.float32),
            pltpu.SemaphoreType.DMA,
        ],
    )


CROWS2 = 32


def _seg_chunk(tab, srcv, dstv, vals, acc_sp, sem_s):
    @plsc.parallel_loop(0, CROWS2, 1, unroll=2)
    def _(j):
        for k in range(ROWL // 16):
            idx = srcv[j, pl.ds(k * 16, 16)]
            vals[pl.ds(j * ROWL + k * 16, 16)] = plsc.load_gather(tab, [idx])
        pltpu.async_copy(vals.at[pl.ds(j * ROWL, ROWL)],
                         acc_sp.at[dstv.at[j]], sem_s, add=True)

    pltpu.make_async_copy(vals, acc_sp.at[pl.ds(0, CROWS2 * ROWL)],
                          sem_s).wait()


def _zero_acc(acc_sp, zbuf, base):
    _fill(zbuf, SLICE, 0.0)
    pltpu.sync_copy(zbuf, acc_sp.at[pl.ds(base, SLICE)])


@functools.lru_cache(maxsize=None)
def _sc_segsum1(nrows):
    rpw = nrows // NW
    nch = rpw // CROWS2
    mesh = plsc.VectorSubcoreMesh(core_axis_name="c", subcore_axis_name="s")

    def body(src_hbm, dst_hbm, g_hbm, out_hbm,
             acc_sp, tab, srcv, dstv, vals, zbuf, sem_s):
        c = lax.axis_index("c")
        s = lax.axis_index("s")
        base = s * SLICE
        _zero_acc(acc_sp, zbuf, base)
        pltpu.sync_copy(g_hbm, tab)
        plsc.subcore_barrier()
        w = s * NC + c

        def chunk(ch, _):
            row0 = w * rpw + ch * CROWS2
            pltpu.sync_copy(src_hbm.at[pl.ds(row0, CROWS2)], srcv)
            pltpu.sync_copy(dst_hbm.at[pl.ds(row0, CROWS2)], dstv)
            _seg_chunk(tab, srcv, dstv, vals, acc_sp, sem_s)
            return 0

        lax.fori_loop(0, nch, chunk, 0)
        plsc.subcore_barrier()
        pltpu.sync_copy(acc_sp.at[pl.ds(base, SLICE)],
                        out_hbm.at[c, pl.ds(base, SLICE)])

    return pl.kernel(
        body,
        out_type=jax.ShapeDtypeStruct((NC, NP), jnp.float32),
        mesh=mesh,
        compiler_params=pltpu.CompilerParams(needs_layout_passes=False),
        scratch_types=[
            pltpu.VMEM_SHARED((NP,), jnp.float32),
            pltpu.VMEM((NP,), jnp.float32),
            pltpu.VMEM((CROWS2, ROWL), jnp.int32),
            pltpu.VMEM((CROWS2, ROWL), jnp.int32),
            pltpu.VMEM((CROWS2 * ROWL,), jnp.float32),
            pltpu.VMEM((SLICE,), jnp.float32),
            pltpu.SemaphoreType.DMA,
        ],
    )


@functools.lru_cache(maxsize=None)
def _sc_segsum2(nrows):
    rpw = nrows // NS
    nch = rpw // CROWS2
    mesh = plsc.VectorSubcoreMesh(core_axis_name="c", subcore_axis_name="s")

    def body(src_hbm, dst_hbm, ga_hbm, gb_hbm, out_hbm,
             acc_sp, tab, srcv, dstv, vals, zbuf, sem_s):
        c = lax.axis_index("c")
        s = lax.axis_index("s")
        base = s * SLICE
        _zero_acc(acc_sp, zbuf, base)

        @pl.when(c == 0)
        def _():
            pltpu.sync_copy(ga_hbm, tab)

        @pl.when(c == 1)
        def _():
            pltpu.sync_copy(gb_hbm, tab)

        plsc.subcore_barrier()

        def chunk(ch, _):
            row0 = s * rpw + ch * CROWS2
            pltpu.sync_copy(src_hbm.at[pl.ds(row0, CROWS2)], srcv)
            pltpu.sync_copy(dst_hbm.at[pl.ds(row0, CROWS2)], dstv)
            _seg_chunk(tab, srcv, dstv, vals, acc_sp, sem_s)
            return 0

        lax.fori_loop(0, nch, chunk, 0)
        plsc.subcore_barrier()
        pltpu.sync_copy(acc_sp.at[pl.ds(base, SLICE)],
                        out_hbm.at[c, pl.ds(base, SLICE)])

    return pl.kernel(
        body,
        out_type=jax.ShapeDtypeStruct((NC, NP), jnp.float32),
        mesh=mesh,
        compiler_params=pltpu.CompilerParams(needs_layout_passes=False),
        scratch_types=[
            pltpu.VMEM_SHARED((NP,), jnp.float32),
            pltpu.VMEM((NP,), jnp.float32),
            pltpu.VMEM((CROWS2, ROWL), jnp.int32),
            pltpu.VMEM((CROWS2, ROWL), jnp.int32),
            pltpu.VMEM((CROWS2 * ROWL,), jnp.float32),
            pltpu.VMEM((SLICE,), jnp.float32),
            pltpu.SemaphoreType.DMA,
        ],
    )


def _tc_prep_body(degp_ref, x_ref, dinv_ref, g1_ref):
    deg = degp_ref[0] + degp_ref[1] + 1.0
    dinv = lax.rsqrt(deg)
    dinv_ref[...] = dinv
    g1_ref[...] = dinv * x_ref[...]


_tc_prep = pl.pallas_call(
    _tc_prep_body,
    out_shape=(jax.ShapeDtypeStruct((NPR, 128), jnp.float32),
               jax.ShapeDtypeStruct((NPR, 128), jnp.float32)),
)


def _tc_mid_body(s_ref, dinv_ref, x_ref, w1_ref, b1_ref, w2_ref,
                 ha_ref, hb_ref, ga_ref, gb_ref):
    dinv = dinv_ref[...]
    t = dinv * (s_ref[0] + s_ref[1]) + dinv * dinv * x_ref[...]
    acc0 = jnp.zeros_like(t)
    acc1 = jnp.zeros_like(t)
    for k in range(16):
        h = jnp.maximum(t * w1_ref[0, k] + b1_ref[0, k], 0.0)
        acc0 = acc0 + h * w2_ref[k, 0]
        acc1 = acc1 + h * w2_ref[k, 1]
    ha_ref[...] = acc0
    hb_ref[...] = acc1
    ga_ref[...] = dinv * acc0
    gb_ref[...] = dinv * acc1


_tc_mid = pl.pallas_call(
    _tc_mid_body,
    out_shape=tuple(jax.ShapeDtypeStruct((NPR, 128), jnp.float32)
                    for _ in range(4)),
)


def _tc_final_body(o_ref, dinv_ref, ha_ref, hb_ref, b2_ref,
                   outa_ref, outb_ref):
    dinv = dinv_ref[...]
    d2 = dinv * dinv
    outa_ref[...] = dinv * o_ref[0] + d2 * ha_ref[...] + b2_ref[0, 0]
    outb_ref[...] = dinv * o_ref[1] + d2 * hb_ref[...] + b2_ref[0, 1]


_tc_final = pl.pallas_call(
    _tc_final_body,
    out_shape=(jax.ShapeDtypeStruct((NPR, 128), jnp.float32),
               jax.ShapeDtypeStruct((NPR, 128), jnp.float32)),
)


def kernel(x, edge_index, W1, b1, W2, b2):
    n = x.shape[0]
    e = edge_index.shape[1]
    chunk_edges = NW * CROWS * ROWL
    ep = ((e + chunk_edges - 1) // chunk_edges) * chunk_edges
    nrows = ep // ROWL

    pad_idx = jnp.full((ep - e,), NP - 1, edge_index.dtype)
    srcp = jnp.concatenate([edge_index[0], pad_idx]).reshape(nrows, ROWL)
    dstp = jnp.concatenate([edge_index[1], pad_idx]).reshape(nrows, ROWL)
    xf = jnp.concatenate(
        [x[:, 0], jnp.zeros((NP - n,), x.dtype)]).reshape(NPR, 128)

    deg2 = _sc_degree(nrows)(dstp).reshape(NC, NPR, 128)
    dinv, g1 = _tc_prep(deg2, xf)
    s2 = _sc_segsum1(nrows)(srcp, dstp, g1.reshape(NP)).reshape(NC, NPR, 128)
    ha, hb, ga, gb = _tc_mid(s2, dinv, xf, W1, b1.reshape(1, 16), W2)
    o2 = _sc_segsum2(nrows)(srcp, dstp, ga.reshape(NP), gb.reshape(NP))
    outa, outb = _tc_final(o2.reshape(NC, NPR, 128),
                           dinv, ha, hb, b2.reshape(1, 2))
    return jnp.stack([outa.reshape(NP)[:n], outb.reshape(NP)[:n]], axis=1)


# speedup vs baseline: 265.9365x; 1.6160x over previous
import functools

import jax
import jax.numpy as jnp
from jax import lax
from jax.experimental import pallas as pl
from jax.experimental.pallas import tpu as pltpu
from jax.experimental.pallas import tpu_sc as plsc

N_NODES = 100000
NP = 100352
NPR = 784
NC, NS = 2, 16
NW = NC * NS
SLICE = NP // NS
ROWL = 128
CR = 32

_SC_PARAMS = pltpu.CompilerParams(needs_layout_passes=False)


def _mesh():
    return plsc.VectorSubcoreMesh(core_axis_name="c", subcore_axis_name="s")


def _fill(ref, n, value):
    v = jnp.full((16,), value, jnp.float32)

    def body(i, _):
        ref[pl.ds(i * 16, 16)] = v
        return 0

    lax.fori_loop(0, n // 16, body, 0)


def _zero_acc(acc_sp, zbuf, base):
    _fill(zbuf, SLICE, 0.0)
    pltpu.sync_copy(zbuf, acc_sp.at[pl.ds(base, SLICE)])


def _gather_chunk(tab, srcv, vals, dstv, acc_sp, sem_s):
    @plsc.parallel_loop(0, CR, 1, unroll=4)
    def _(j):
        for k in range(ROWL // 16):
            idx = srcv[j, pl.ds(k * 16, 16)]
            vals[pl.ds(j * ROWL + k * 16, 16)] = plsc.load_gather(tab, [idx])
        pltpu.async_copy(vals.at[pl.ds(j * ROWL, ROWL)],
                         acc_sp.at[dstv.at[j]], sem_s, add=True)


def _drain(vals, acc_sp, sem_s):
    pltpu.make_async_copy(vals, acc_sp.at[pl.ds(0, CR * ROWL)], sem_s).wait()


def _row_range(w, nw, nrows):
    units = nrows // 8
    base = ((w * units) // nw) * 8
    end = (((w + 1) * units) // nw) * 8
    cnt = end - base
    return base, end, cnt


def _edge_pass(src_hbm, dst_hbm, tab, srcv, dstv, vals, acc_sp, sem_s,
               w, nw, nrows, gather):
    base, end, cnt = _row_range(w, nw, nrows)
    nch = cnt // CR
    rem = cnt - nch * CR

    def chunk(ch, _):
        row0 = base + ch * CR
        if gather:
            pltpu.sync_copy(src_hbm.at[pl.ds(row0, CR)], srcv)
        pltpu.sync_copy(dst_hbm.at[pl.ds(row0, CR)], dstv)
        if gather:
            _gather_chunk(tab, srcv, vals, dstv, acc_sp, sem_s)
        else:
            for j in range(CR):
                pltpu.async_copy(vals.at[pl.ds(j * ROWL, ROWL)],
                                 acc_sp.at[dstv.at[j]], sem_s, add=True)
        _drain(vals, acc_sp, sem_s)
        return 0

    lax.fori_loop(0, nch, chunk, 0)

    @pl.when(rem > 0)
    def _():
        row0 = end - CR
        if gather:
            pltpu.sync_copy(src_hbm.at[pl.ds(row0, CR)], srcv)
        pltpu.sync_copy(dst_hbm.at[pl.ds(row0, CR)], dstv)
        if gather:

            @plsc.parallel_loop(0, CR, 1, unroll=4)
            def _(j):
                for k in range(ROWL // 16):
                    idx = srcv[j, pl.ds(k * 16, 16)]
                    vals[pl.ds(j * ROWL + k * 16, 16)] = (
                        plsc.load_gather(tab, [idx]))

        for j in range(CR):

            @pl.when(j >= CR - rem)
            def _():
                pltpu.sync_copy(vals.at[pl.ds(j * ROWL, ROWL)],
                                acc_sp.at[dstv.at[j]], add=True)


@functools.lru_cache(maxsize=None)
def _sc_degree(nrows):
    def body(dst_hbm, out_hbm, acc_sp, dstv, vals, zbuf, sem_s):
        c = lax.axis_index("c")
        s = lax.axis_index("s")
        base = s * SLICE
        _fill(vals, CR * ROWL, 1.0)
        _zero_acc(acc_sp, zbuf, base)
        plsc.subcore_barrier()
        _edge_pass(None, dst_hbm, None, None, dstv, vals, acc_sp, sem_s,
                   s * NC + c, NW, nrows, gather=False)
        plsc.subcore_barrier()
        pltpu.sync_copy(acc_sp.at[pl.ds(base, SLICE)],
                        out_hbm.at[c, pl.ds(base, SLICE)])

    return pl.kernel(
        body,
        out_type=jax.ShapeDtypeStruct((NC, NP), jnp.float32),
        mesh=_mesh(),
        compiler_params=_SC_PARAMS,
        scratch_types=[
            pltpu.VMEM_SHARED((NP,), jnp.float32),
            pltpu.VMEM((CR, ROWL), jnp.int32),
            pltpu.VMEM((CR * ROWL,), jnp.float32),
            pltpu.VMEM((SLICE,), jnp.float32),
            pltpu.SemaphoreType.DMA,
        ],
    )


@functools.lru_cache(maxsize=None)
def _sc_segsum1(nrows):
    def body(src_hbm, dst_hbm, g_hbm, out_hbm,
             acc_sp, tab, srcv, dstv, vals, zbuf, sem_s):
        c = lax.axis_index("c")
        s = lax.axis_index("s")
        base = s * SLICE
        _zero_acc(acc_sp, zbuf, base)
        pltpu.sync_copy(g_hbm, tab)
        plsc.subcore_barrier()
        _edge_pass(src_hbm, dst_hbm, tab, srcv, dstv, vals, acc_sp, sem_s,
                   s * NC + c, NW, nrows, gather=True)
        plsc.subcore_barrier()
        pltpu.sync_copy(acc_sp.at[pl.ds(base, SLICE)],
                        out_hbm.at[c, pl.ds(base, SLICE)])

    return pl.kernel(
        body,
        out_type=jax.ShapeDtypeStruct((NC, NP), jnp.float32),
        mesh=_mesh(),
        compiler_params=_SC_PARAMS,
        scratch_types=[
            pltpu.VMEM_SHARED((NP,), jnp.float32),
            pltpu.VMEM((NP,), jnp.float32),
            pltpu.VMEM((CR, ROWL), jnp.int32),
            pltpu.VMEM((CR, ROWL), jnp.int32),
            pltpu.VMEM((CR * ROWL,), jnp.float32),
            pltpu.VMEM((SLICE,), jnp.float32),
            pltpu.SemaphoreType.DMA,
        ],
    )


@functools.lru_cache(maxsize=None)
def _sc_segsum2(nrows):
    def body(src_hbm, dst_hbm, ga_hbm, gb_hbm, out_hbm,
             acc_sp, tab, srcv, dstv, vals, zbuf, sem_s):
        c = lax.axis_index("c")
        s = lax.axis_index("s")
        base = s * SLICE
        _zero_acc(acc_sp, zbuf, base)

        @pl.when(c == 0)
        def _():
            pltpu.sync_copy(ga_hbm, tab)

        @pl.when(c == 1)
        def _():
            pltpu.sync_copy(gb_hbm, tab)

        plsc.subcore_barrier()
        _edge_pass(src_hbm, dst_hbm, tab, srcv, dstv, vals, acc_sp, sem_s,
                   s, NS, nrows, gather=True)
        plsc.subcore_barrier()
        pltpu.sync_copy(acc_sp.at[pl.ds(base, SLICE)],
                        out_hbm.at[c, pl.ds(base, SLICE)])

    return pl.kernel(
        body,
        out_type=jax.ShapeDtypeStruct((NC, NP), jnp.float32),
        mesh=_mesh(),
        compiler_params=_SC_PARAMS,
        scratch_types=[
            pltpu.VMEM_SHARED((NP,), jnp.float32),
            pltpu.VMEM((NP,), jnp.float32),
            pltpu.VMEM((CR, ROWL), jnp.int32),
            pltpu.VMEM((CR, ROWL), jnp.int32),
            pltpu.VMEM((CR * ROWL,), jnp.float32),
            pltpu.VMEM((SLICE,), jnp.float32),
            pltpu.SemaphoreType.DMA,
        ],
    )


def _tc_prep_body(degp_ref, x_ref, dinv_ref, g1_ref):
    deg = degp_ref[0] + degp_ref[1] + 1.0
    dinv = lax.rsqrt(deg)
    dinv_ref[...] = dinv
    g1_ref[...] = dinv * x_ref[...]


_tc_prep = pl.pallas_call(
    _tc_prep_body,
    out_shape=(jax.ShapeDtypeStruct((NPR, 128), jnp.float32),
               jax.ShapeDtypeStruct((NPR, 128), jnp.float32)),
)


def _tc_mid_body(s_ref, dinv_ref, x_ref, w1_ref, b1_ref, w2_ref,
                 ha_ref, hb_ref, ga_ref, gb_ref):
    dinv = dinv_ref[...]
    t = dinv * (s_ref[0] + s_ref[1]) + dinv * dinv * x_ref[...]
    acc0 = jnp.zeros_like(t)
    acc1 = jnp.zeros_like(t)
    for k in range(16):
        h = jnp.maximum(t * w1_ref[0, k] + b1_ref[0, k], 0.0)
        acc0 = acc0 + h * w2_ref[k, 0]
        acc1 = acc1 + h * w2_ref[k, 1]
    ha_ref[...] = acc0
    hb_ref[...] = acc1
    ga_ref[...] = dinv * acc0
    gb_ref[...] = dinv * acc1


_tc_mid = pl.pallas_call(
    _tc_mid_body,
    out_shape=tuple(jax.ShapeDtypeStruct((NPR, 128), jnp.float32)
                    for _ in range(4)),
)


def _tc_final_body(o_ref, dinv_ref, ha_ref, hb_ref, b2_ref,
                   outa_ref, outb_ref):
    dinv = dinv_ref[...]
    d2 = dinv * dinv
    outa_ref[...] = dinv * o_ref[0] + d2 * ha_ref[...] + b2_ref[0, 0]
    outb_ref[...] = dinv * o_ref[1] + d2 * hb_ref[...] + b2_ref[0, 1]


_tc_final = pl.pallas_call(
    _tc_final_body,
    out_shape=(jax.ShapeDtypeStruct((NPR, 128), jnp.float32),
               jax.ShapeDtypeStruct((NPR, 128), jnp.float32)),
)


def kernel(x, edge_index, W1, b1, W2, b2):
    n = x.shape[0]
    e = edge_index.shape[1]
    if e % (8 * ROWL):
        pad = 8 * ROWL - e % (8 * ROWL)
        edge_index = jnp.concatenate(
            [edge_index, jnp.full((2, pad), n, edge_index.dtype)], axis=1)
        e += pad
    nrows = e // ROWL
    srcp = edge_index[0].reshape(nrows, ROWL)
    dstp = edge_index[1].reshape(nrows, ROWL)
    xf = jnp.concatenate(
        [x[:, 0], jnp.zeros((NP - n,), x.dtype)]).reshape(NPR, 128)

    deg2 = _sc_degree(nrows)(dstp).reshape(NC, NPR, 128)
    dinv, g1 = _tc_prep(deg2, xf)
    s2 = _sc_segsum1(nrows)(srcp, dstp, g1.reshape(NP)).reshape(NC, NPR, 128)
    ha, hb, ga, gb = _tc_mid(s2, dinv, xf, W1, b1.reshape(1, 16), W2)
    o2 = _sc_segsum2(nrows)(srcp, dstp, ga.reshape(NP), gb.reshape(NP))
    outa, outb = _tc_final(o2.reshape(NC, NPR, 128),
                           dinv, ha, hb, b2.reshape(1, 2))
    return jnp.stack([outa.reshape(NP)[:n], outb.reshape(NP)[:n]], axis=1)


# speedup vs baseline: 296.6695x; 1.1156x over previous
import functools

import jax
import jax.numpy as jnp
from jax import lax
from jax.experimental import pallas as pl
from jax.experimental.pallas import tpu as pltpu
from jax.experimental.pallas import tpu_sc as plsc

N_NODES = 100000
NP = 100352
NPR = 784
NC, NS = 2, 16
NW = NC * NS
SLICE = NP // NS
ROWL = 128
CR = 48

_SC_PARAMS = pltpu.CompilerParams(needs_layout_passes=False)


def _mesh():
    return plsc.VectorSubcoreMesh(core_axis_name="c", subcore_axis_name="s")


def _fill(ref, n, value):
    v = jnp.full((16,), value, jnp.float32)

    def body(i, _):
        ref[pl.ds(i * 16, 16)] = v
        return 0

    lax.fori_loop(0, n // 16, body, 0)


def _zero_acc(acc_sp, vals, base):
    _fill(vals, SLICE, 0.0)
    pltpu.sync_copy(vals.at[pl.ds(0, SLICE)], acc_sp.at[pl.ds(base, SLICE)])


def _gather_chunk(tab, srcv, vals, dstv, acc_sp, sem_s):
    @plsc.parallel_loop(0, CR, 1, unroll=8)
    def _(j):
        for k in range(ROWL // 16):
            idx = srcv[j, pl.ds(k * 16, 16)]
            vals[pl.ds(j * ROWL + k * 16, 16)] = plsc.load_gather(tab, [idx])
        pltpu.async_copy(vals.at[pl.ds(j * ROWL, ROWL)],
                         acc_sp.at[dstv.at[j]], sem_s, add=True)


def _drain(vals, acc_sp, sem_s):
    pltpu.make_async_copy(vals, acc_sp.at[pl.ds(0, CR * ROWL)], sem_s).wait()


def _row_range(w, nw, nrows):
    units = nrows // 8
    base = ((w * units) // nw) * 8
    end = (((w + 1) * units) // nw) * 8
    cnt = end - base
    return base, end, cnt


def _edge_pass(src_hbm, dst_hbm, tab, srcv, dstv, vals, acc_sp, sem_s,
               w, nw, nrows, gather):
    base, end, cnt = _row_range(w, nw, nrows)
    nch = cnt // CR
    rem = cnt - nch * CR

    def chunk(ch, _):
        row0 = base + ch * CR
        if gather:
            pltpu.sync_copy(src_hbm.at[pl.ds(row0, CR)], srcv)
        pltpu.sync_copy(dst_hbm.at[pl.ds(row0, CR)], dstv)
        if gather:
            _gather_chunk(tab, srcv, vals, dstv, acc_sp, sem_s)
        else:
            for j in range(CR):
                pltpu.async_copy(vals.at[pl.ds(j * ROWL, ROWL)],
                                 acc_sp.at[dstv.at[j]], sem_s, add=True)
        _drain(vals, acc_sp, sem_s)
        return 0

    lax.fori_loop(0, nch, chunk, 0)

    @pl.when(rem > 0)
    def _():
        row0 = end - CR
        if gather:
            pltpu.sync_copy(src_hbm.at[pl.ds(row0, CR)], srcv)
        pltpu.sync_copy(dst_hbm.at[pl.ds(row0, CR)], dstv)
        if gather:

            @plsc.parallel_loop(0, CR, 1, unroll=4)
            def _(j):
                for k in range(ROWL // 16):
                    idx = srcv[j, pl.ds(k * 16, 16)]
                    vals[pl.ds(j * ROWL + k * 16, 16)] = (
                        plsc.load_gather(tab, [idx]))

        for j in range(CR):

            @pl.when(j >= CR - rem)
            def _():
                pltpu.sync_copy(vals.at[pl.ds(j * ROWL, ROWL)],
                                acc_sp.at[dstv.at[j]], add=True)


@functools.lru_cache(maxsize=None)
def _sc_degree(nrows):
    def body(dst_hbm, out_hbm, acc_sp, dstv, vals, sem_s):
        c = lax.axis_index("c")
        s = lax.axis_index("s")
        base = s * SLICE
        _zero_acc(acc_sp, vals, base)
        _fill(vals, CR * ROWL, 1.0)
        plsc.subcore_barrier()
        _edge_pass(None, dst_hbm, None, None, dstv, vals, acc_sp, sem_s,
                   s * NC + c, NW, nrows, gather=False)
        plsc.subcore_barrier()
        pltpu.sync_copy(acc_sp.at[pl.ds(base, SLICE)],
                        out_hbm.at[c, pl.ds(base, SLICE)])

    return pl.kernel(
        body,
        out_type=jax.ShapeDtypeStruct((NC, NP), jnp.float32),
        mesh=_mesh(),
        compiler_params=_SC_PARAMS,
        scratch_types=[
            pltpu.VMEM_SHARED((NP,), jnp.float32),
            pltpu.VMEM((CR, ROWL), jnp.int32),
            pltpu.VMEM((CR * ROWL,), jnp.float32),
            pltpu.SemaphoreType.DMA,
        ],
    )


@functools.lru_cache(maxsize=None)
def _sc_segsum1(nrows):
    def body(src_hbm, dst_hbm, g_hbm, out_hbm,
             acc_sp, tab, srcv, dstv, vals, sem_s):
        c = lax.axis_index("c")
        s = lax.axis_index("s")
        base = s * SLICE
        _zero_acc(acc_sp, vals, base)
        pltpu.sync_copy(g_hbm, tab)
        plsc.subcore_barrier()
        _edge_pass(src_hbm, dst_hbm, tab, srcv, dstv, vals, acc_sp, sem_s,
                   s * NC + c, NW, nrows, gather=True)
        plsc.subcore_barrier()
        pltpu.sync_copy(acc_sp.at[pl.ds(base, SLICE)],
                        out_hbm.at[c, pl.ds(base, SLICE)])

    return pl.kernel(
        body,
        out_type=jax.ShapeDtypeStruct((NC, NP), jnp.float32),
        mesh=_mesh(),
        compiler_params=_SC_PARAMS,
        scratch_types=[
            pltpu.VMEM_SHARED((NP,), jnp.float32),
            pltpu.VMEM((NP,), jnp.float32),
            pltpu.VMEM((CR, ROWL), jnp.int32),
            pltpu.VMEM((CR, ROWL), jnp.int32),
            pltpu.VMEM((CR * ROWL,), jnp.float32),
            pltpu.SemaphoreType.DMA,
        ],
    )


@functools.lru_cache(maxsize=None)
def _sc_segsum2(nrows):
    def body(src_hbm, dst_hbm, ga_hbm, gb_hbm, out_hbm,
             acc_sp, tab, srcv, dstv, vals, sem_s):
        c = lax.axis_index("c")
        s = lax.axis_index("s")
        base = s * SLICE
        _zero_acc(acc_sp, vals, base)

        @pl.when(c == 0)
        def _():
            pltpu.sync_copy(ga_hbm, tab)

        @pl.when(c == 1)
        def _():
            pltpu.sync_copy(gb_hbm, tab)

        plsc.subcore_barrier()
        _edge_pass(src_hbm, dst_hbm, tab, srcv, dstv, vals, acc_sp, sem_s,
                   s, NS, nrows, gather=True)
        plsc.subcore_barrier()
        pltpu.sync_copy(acc_sp.at[pl.ds(base, SLICE)],
                        out_hbm.at[c, pl.ds(base, SLICE)])

    return pl.kernel(
        body,
        out_type=jax.ShapeDtypeStruct((NC, NP), jnp.float32),
        mesh=_mesh(),
        compiler_params=_SC_PARAMS,
        scratch_types=[
            pltpu.VMEM_SHARED((NP,), jnp.float32),
            pltpu.VMEM((NP,), jnp.float32),
            pltpu.VMEM((CR, ROWL), jnp.int32),
            pltpu.VMEM((CR, ROWL), jnp.int32),
            pltpu.VMEM((CR * ROWL,), jnp.float32),
            pltpu.SemaphoreType.DMA,
        ],
    )


def _tc_prep_body(degp_ref, x_ref, dinv_ref, g1_ref):
    deg = degp_ref[0] + degp_ref[1] + 1.0
    dinv = lax.rsqrt(deg)
    dinv_ref[...] = dinv
    g1_ref[...] = dinv * x_ref[...]


_tc_prep = pl.pallas_call(
    _tc_prep_body,
    out_shape=(jax.ShapeDtypeStruct((NPR, 128), jnp.float32),
               jax.ShapeDtypeStruct((NPR, 128), jnp.float32)),
)


def _tc_mid_body(s_ref, dinv_ref, x_ref, w1_ref, b1_ref, w2_ref,
                 ha_ref, hb_ref, ga_ref, gb_ref):
    dinv = dinv_ref[...]
    t = dinv * (s_ref[0] + s_ref[1]) + dinv * dinv * x_ref[...]
    acc0 = jnp.zeros_like(t)
    acc1 = jnp.zeros_like(t)
    for k in range(16):
        h = jnp.maximum(t * w1_ref[0, k] + b1_ref[0, k], 0.0)
        acc0 = acc0 + h * w2_ref[k, 0]
        acc1 = acc1 + h * w2_ref[k, 1]
    ha_ref[...] = acc0
    hb_ref[...] = acc1
    ga_ref[...] = dinv * acc0
    gb_ref[...] = dinv * acc1


_tc_mid = pl.pallas_call(
    _tc_mid_body,
    out_shape=tuple(jax.ShapeDtypeStruct((NPR, 128), jnp.float32)
                    for _ in range(4)),
)


def _tc_final_body(o_ref, dinv_ref, ha_ref, hb_ref, b2_ref,
                   outa_ref, outb_ref):
    dinv = dinv_ref[...]
    d2 = dinv * dinv
    outa_ref[...] = dinv * o_ref[0] + d2 * ha_ref[...] + b2_ref[0, 0]
    outb_ref[...] = dinv * o_ref[1] + d2 * hb_ref[...] + b2_ref[0, 1]


_tc_final = pl.pallas_call(
    _tc_final_body,
    out_shape=(jax.ShapeDtypeStruct((NPR, 128), jnp.float32),
               jax.ShapeDtypeStruct((NPR, 128), jnp.float32)),
)


def kernel(x, edge_index, W1, b1, W2, b2):
    n = x.shape[0]
    e = edge_index.shape[1]
    if e % (8 * ROWL):
        pad = 8 * ROWL - e % (8 * ROWL)
        edge_index = jnp.concatenate(
            [edge_index, jnp.full((2, pad), n, edge_index.dtype)], axis=1)
        e += pad
    nrows = e // ROWL
    srcp = edge_index[0].reshape(nrows, ROWL)
    dstp = edge_index[1].reshape(nrows, ROWL)
    xf = jnp.concatenate(
        [x[:, 0], jnp.zeros((NP - n,), x.dtype)]).reshape(NPR, 128)

    deg2 = _sc_degree(nrows)(dstp).reshape(NC, NPR, 128)
    dinv, g1 = _tc_prep(deg2, xf)
    s2 = _sc_segsum1(nrows)(srcp, dstp, g1.reshape(NP)).reshape(NC, NPR, 128)
    ha, hb, ga, gb = _tc_mid(s2, dinv, xf, W1, b1.reshape(1, 16), W2)
    o2 = _sc_segsum2(nrows)(srcp, dstp, ga.reshape(NP), gb.reshape(NP))
    outa, outb = _tc_final(o2.reshape(NC, NPR, 128),
                           dinv, ha, hb, b2.reshape(1, 2))
    return jnp.stack([outa.reshape(NP)[:n], outb.reshape(NP)[:n]], axis=1)


# speedup vs baseline: 361.1479x; 1.2173x over previous
import functools

import jax
import jax.numpy as jnp
from jax import lax
from jax.experimental import pallas as pl
from jax.experimental.pallas import tpu as pltpu
from jax.experimental.pallas import tpu_sc as plsc

N_NODES = 100000
NP = 100352
NPR = 784
NC, NS = 2, 16
NW = NC * NS
SLICE = NP // NS
CHUNK = 2048

_SC_PARAMS = pltpu.CompilerParams(needs_layout_passes=False)


def _mesh():
    return plsc.VectorSubcoreMesh(core_axis_name="c", subcore_axis_name="s")


def _fill(ref, n, value):
    v = jnp.full((16,), value, jnp.float32)

    def body(i, _):
        ref[pl.ds(i * 16, 16)] = v
        return 0

    lax.fori_loop(0, n // 16, body, 0)


def _zero_acc(acc_sp, zsrc, base):
    _fill(zsrc, CHUNK, 0.0)
    for i in range(SLICE // CHUNK):
        pltpu.sync_copy(zsrc, acc_sp.at[pl.ds(base + i * CHUNK, CHUNK)])
    rem = SLICE % CHUNK
    if rem:
        pltpu.sync_copy(zsrc.at[pl.ds(0, rem)],
                        acc_sp.at[pl.ds(base + SLICE - rem, rem)])


def _chunk_range(w, nw, nchunks):
    lo = (w * nchunks) // nw
    hi = ((w + 1) * nchunks) // nw
    return lo, hi


def _edge_pass(src_hbm, dst_hbm, tab, sv, dv, vw, acc_sp, sem_i, sem_s,
               w, nw, nchunks, gather):
    lo, hi = _chunk_range(w, nw, nchunks)
    total = hi - lo

    def vwb(b):
        return vw if not gather else vw[b]

    def prefetch(i, b):
        e0 = i * CHUNK
        if gather:
            pltpu.async_copy(src_hbm.at[pl.ds(e0, CHUNK)], sv[b], sem_i[b])
        pltpu.async_copy(dst_hbm.at[pl.ds(e0, CHUNK)], dv[b], sem_i[b])

    def wait_prefetch(i, b):
        e0 = i * CHUNK
        if gather:
            pltpu.make_async_copy(src_hbm.at[pl.ds(e0, CHUNK)], sv[b],
                                  sem_i[b]).wait()
        pltpu.make_async_copy(dst_hbm.at[pl.ds(e0, CHUNK)], dv[b],
                              sem_i[b]).wait()

    @pl.when(total > 0)
    def _():
        prefetch(lo, 0)

    def triple(p, _):
        for b in range(3):
            i = lo + p * 3 + b

            @pl.when(i < hi)
            def _():
                wait_prefetch(i, b)
                if gather:

                    @plsc.parallel_loop(0, CHUNK // 16, 1, unroll=8)
                    def _(t):
                        idx = sv[b][pl.ds(t * 16, 16)]
                        vw[b][pl.ds(t * 16, 16)] = (
                            plsc.load_gather(tab, [idx]))

                pltpu.async_copy(vwb(b), acc_sp.at[dv[b]], sem_s[b],
                                 add=True)
                nb = (b + 1) % 3

                @pl.when(i > lo + 1)
                def _():
                    pltpu.make_async_copy(vwb(nb), acc_sp.at[dv[nb]],
                                          sem_s[nb]).wait()

                @pl.when(i + 1 < hi)
                def _():
                    prefetch(i + 1, nb)
        return 0

    lax.fori_loop(0, (total + 2) // 3, triple, 0)

    for b in range(3):

        @pl.when(jnp.logical_and(total > 0, (total - 1) % 3 == b))
        def _():
            pltpu.make_async_copy(vwb(b), acc_sp.at[dv[b]], sem_s[b]).wait()

        @pl.when(jnp.logical_and(total > 1, (total - 2) % 3 == b))
        def _():
            pltpu.make_async_copy(vwb(b), acc_sp.at[dv[b]], sem_s[b]).wait()


@functools.lru_cache(maxsize=None)
def _sc_degree(nchunks):
    def body(dst_hbm, out_hbm, acc_sp, dv0, dv1, dv2, vw,
             si0, si1, si2, ss0, ss1, ss2):
        c = lax.axis_index("c")
        s = lax.axis_index("s")
        base = s * SLICE
        _zero_acc(acc_sp, vw, base)
        _fill(vw, CHUNK, 1.0)
        plsc.subcore_barrier()
        _edge_pass(None, dst_hbm, None, None, (dv0, dv1, dv2), vw,
                   acc_sp, (si0, si1, si2), (ss0, ss1, ss2),
                   s * NC + c, NW, nchunks, gather=False)
        plsc.subcore_barrier()
        pltpu.sync_copy(acc_sp.at[pl.ds(base, SLICE)],
                        out_hbm.at[c, pl.ds(base, SLICE)])

    return pl.kernel(
        body,
        out_type=jax.ShapeDtypeStruct((NC, NP), jnp.float32),
        mesh=_mesh(),
        compiler_params=_SC_PARAMS,
        scratch_types=[
            pltpu.VMEM_SHARED((NP,), jnp.float32),
            pltpu.VMEM((CHUNK,), jnp.int32),
            pltpu.VMEM((CHUNK,), jnp.int32),
            pltpu.VMEM((CHUNK,), jnp.int32),
            pltpu.VMEM((CHUNK,), jnp.float32),
        ] + [pltpu.SemaphoreType.DMA] * 6,
    )


_SEG_SCRATCH = (
    [pltpu.VMEM_SHARED((NP,), jnp.float32),
     pltpu.VMEM((NP,), jnp.float32)]
    + [pltpu.VMEM((CHUNK,), jnp.int32)] * 6
    + [pltpu.VMEM((CHUNK,), jnp.float32)] * 3
    + [pltpu.SemaphoreType.DMA] * 6
)


@functools.lru_cache(maxsize=None)
def _sc_segsum1(nchunks):
    def body(src_hbm, dst_hbm, g_hbm, out_hbm, acc_sp, tab,
             sv0, sv1, sv2, dv0, dv1, dv2, vw0, vw1, vw2,
             si0, si1, si2, ss0, ss1, ss2):
        c = lax.axis_index("c")
        s = lax.axis_index("s")
        base = s * SLICE
        _zero_acc(acc_sp, vw0, base)
        pltpu.sync_copy(g_hbm, tab)
        plsc.subcore_barrier()
        _edge_pass(src_hbm, dst_hbm, tab, (sv0, sv1, sv2),
                   (dv0, dv1, dv2), (vw0, vw1, vw2), acc_sp,
                   (si0, si1, si2), (ss0, ss1, ss2),
                   s * NC + c, NW, nchunks, gather=True)
        plsc.subcore_barrier()
        pltpu.sync_copy(acc_sp.at[pl.ds(base, SLICE)],
                        out_hbm.at[c, pl.ds(base, SLICE)])

    return pl.kernel(
        body,
        out_type=jax.ShapeDtypeStruct((NC, NP), jnp.float32),
        mesh=_mesh(),
        compiler_params=_SC_PARAMS,
        scratch_types=list(_SEG_SCRATCH),
    )


@functools.lru_cache(maxsize=None)
def _sc_segsum2(nchunks):
    def body(src_hbm, dst_hbm, ga_hbm, gb_hbm, out_hbm, acc_sp, tab,
             sv0, sv1, sv2, dv0, dv1, dv2, vw0, vw1, vw2,
             si0, si1, si2, ss0, ss1, ss2):
        c = lax.axis_index("c")
        s = lax.axis_index("s")
        base = s * SLICE
        _zero_acc(acc_sp, vw0, base)

        @pl.when(c == 0)
        def _():
            pltpu.sync_copy(ga_hbm, tab)

        @pl.when(c == 1)
        def _():
            pltpu.sync_copy(gb_hbm, tab)

        plsc.subcore_barrier()
        _edge_pass(src_hbm, dst_hbm, tab, (sv0, sv1, sv2),
                   (dv0, dv1, dv2), (vw0, vw1, vw2), acc_sp,
                   (si0, si1, si2), (ss0, ss1, ss2),
                   s, NS, nchunks, gather=True)
        plsc.subcore_barrier()
        pltpu.sync_copy(acc_sp.at[pl.ds(base, SLICE)],
                        out_hbm.at[c, pl.ds(base, SLICE)])

    return pl.kernel(
        body,
        out_type=jax.ShapeDtypeStruct((NC, NP), jnp.float32),
        mesh=_mesh(),
        compiler_params=_SC_PARAMS,
        scratch_types=list(_SEG_SCRATCH),
    )


def _tc_prep_body(degp_ref, x_ref, dinv_ref, g1_ref):
    deg = degp_ref[0] + degp_ref[1] + 1.0
    dinv = lax.rsqrt(deg)
    dinv_ref[...] = dinv
    g1_ref[...] = dinv * x_ref[...]


_tc_prep = pl.pallas_call(
    _tc_prep_body,
    out_shape=(jax.ShapeDtypeStruct((NPR, 128), jnp.float32),
               jax.ShapeDtypeStruct((NPR, 128), jnp.float32)),
)


def _tc_mid_body(s_ref, dinv_ref, x_ref, w1_ref, b1_ref, w2_ref,
                 ha_ref, hb_ref, ga_ref, gb_ref):
    dinv = dinv_ref[...]
    t = dinv * (s_ref[0] + s_ref[1]) + dinv * dinv * x_ref[...]
    acc0 = jnp.zeros_like(t)
    acc1 = jnp.zeros_like(t)
    for k in range(16):
        h = jnp.maximum(t * w1_ref[0, k] + b1_ref[0, k], 0.0)
        acc0 = acc0 + h * w2_ref[k, 0]
        acc1 = acc1 + h * w2_ref[k, 1]
    ha_ref[...] = acc0
    hb_ref[...] = acc1
    ga_ref[...] = dinv * acc0
    gb_ref[...] = dinv * acc1


_tc_mid = pl.pallas_call(
    _tc_mid_body,
    out_shape=tuple(jax.ShapeDtypeStruct((NPR, 128), jnp.float32)
                    for _ in range(4)),
)


def _tc_final_body(o_ref, dinv_ref, ha_ref, hb_ref, b2_ref,
                   outa_ref, outb_ref):
    dinv = dinv_ref[...]
    d2 = dinv * dinv
    outa_ref[...] = dinv * o_ref[0] + d2 * ha_ref[...] + b2_ref[0, 0]
    outb_ref[...] = dinv * o_ref[1] + d2 * hb_ref[...] + b2_ref[0, 1]


_tc_final = pl.pallas_call(
    _tc_final_body,
    out_shape=(jax.ShapeDtypeStruct((NPR, 128), jnp.float32),
               jax.ShapeDtypeStruct((NPR, 128), jnp.float32)),
)


def kernel(x, edge_index, W1, b1, W2, b2):
    n = x.shape[0]
    e = edge_index.shape[1]
    if e % CHUNK:
        pad = CHUNK - e % CHUNK
        edge_index = jnp.concatenate(
            [edge_index, jnp.full((2, pad), n, edge_index.dtype)], axis=1)
        e += pad
    nchunks = e // CHUNK
    src = edge_index[0]
    dst = edge_index[1]
    xf = jnp.concatenate(
        [x[:, 0], jnp.zeros((NP - n,), x.dtype)]).reshape(NPR, 128)

    deg2 = _sc_degree(nchunks)(dst).reshape(NC, NPR, 128)
    dinv, g1 = _tc_prep(deg2, xf)
    s2 = _sc_segsum1(nchunks)(src, dst, g1.reshape(NP)).reshape(NC, NPR, 128)
    ha, hb, ga, gb = _tc_mid(s2, dinv, xf, W1, b1.reshape(1, 16), W2)
    o2 = _sc_segsum2(nchunks)(src, dst, ga.reshape(NP), gb.reshape(NP))
    outa, outb = _tc_final(o2.reshape(NC, NPR, 128),
                           dinv, ha, hb, b2.reshape(1, 2))
    return jnp.stack([outa.reshape(NP)[:n], outb.reshape(NP)[:n]], axis=1)


# speedup vs baseline: 361.8883x; 1.0020x over previous
import functools

import jax
import jax.numpy as jnp
from jax import lax
from jax.experimental import pallas as pl
from jax.experimental.pallas import tpu as pltpu
from jax.experimental.pallas import tpu_sc as plsc

N_NODES = 100000
NP = 100352
NPR = 784
NC, NS = 2, 16
NW = NC * NS
SLICE = NP // NS
CHUNK = 2048

_SC_PARAMS = pltpu.CompilerParams(needs_layout_passes=False)


def _mesh():
    return plsc.VectorSubcoreMesh(core_axis_name="c", subcore_axis_name="s")


def _fill(ref, n, value):
    v = jnp.full((16,), value, jnp.float32)

    def body(i, _):
        ref[pl.ds(i * 16, 16)] = v
        return 0

    lax.fori_loop(0, n // 16, body, 0)


def _zero_acc(acc_sp, zsrc, base):
    _fill(zsrc, CHUNK, 0.0)
    for i in range(SLICE // CHUNK):
        pltpu.sync_copy(zsrc, acc_sp.at[pl.ds(base + i * CHUNK, CHUNK)])
    rem = SLICE % CHUNK
    if rem:
        pltpu.sync_copy(zsrc.at[pl.ds(0, rem)],
                        acc_sp.at[pl.ds(base + SLICE - rem, rem)])


def _chunk_range(w, nw, nchunks):
    lo = (w * nchunks) // nw
    hi = ((w + 1) * nchunks) // nw
    return lo, hi


def _edge_pass(src_hbm, dst_hbm, tab, sv, dv, vw, acc_sp, sem_i, sem_s,
               w, nw, nchunks, gather):
    lo, hi = _chunk_range(w, nw, nchunks)
    total = hi - lo

    def vwb(b):
        return vw if not gather else vw[b]

    def prefetch(i, b):
        e0 = i * CHUNK
        if gather:
            pltpu.async_copy(src_hbm.at[pl.ds(e0, CHUNK)], sv[b], sem_i[b])
        pltpu.async_copy(dst_hbm.at[pl.ds(e0, CHUNK)], dv[b], sem_i[b])

    def wait_prefetch(i, b):
        e0 = i * CHUNK
        if gather:
            pltpu.make_async_copy(src_hbm.at[pl.ds(e0, CHUNK)], sv[b],
                                  sem_i[b]).wait()
        pltpu.make_async_copy(dst_hbm.at[pl.ds(e0, CHUNK)], dv[b],
                              sem_i[b]).wait()

    @pl.when(total > 0)
    def _():
        prefetch(lo, 0)

    def triple(p, _):
        for b in range(3):
            i = lo + p * 3 + b

            @pl.when(i < hi)
            def _():
                wait_prefetch(i, b)
                if gather:

                    @plsc.parallel_loop(0, CHUNK // 16, 1, unroll=16)
                    def _(t):
                        idx = sv[b][pl.ds(t * 16, 16)]
                        vw[b][pl.ds(t * 16, 16)] = (
                            plsc.load_gather(tab, [idx]))

                pltpu.async_copy(vwb(b), acc_sp.at[dv[b]], sem_s[b],
                                 add=True)
                nb = (b + 1) % 3

                @pl.when(i > lo + 1)
                def _():
                    pltpu.make_async_copy(vwb(nb), acc_sp.at[dv[nb]],
                                          sem_s[nb]).wait()

                @pl.when(i + 1 < hi)
                def _():
                    prefetch(i + 1, nb)
        return 0

    lax.fori_loop(0, (total + 2) // 3, triple, 0)

    for b in range(3):

        @pl.when(jnp.logical_and(total > 0, (total - 1) % 3 == b))
        def _():
            pltpu.make_async_copy(vwb(b), acc_sp.at[dv[b]], sem_s[b]).wait()

        @pl.when(jnp.logical_and(total > 1, (total - 2) % 3 == b))
        def _():
            pltpu.make_async_copy(vwb(b), acc_sp.at[dv[b]], sem_s[b]).wait()


@functools.lru_cache(maxsize=None)
def _sc_degree(nchunks):
    def body(dst_hbm, out_hbm, acc_sp, dv0, dv1, dv2, vw,
             si0, si1, si2, ss0, ss1, ss2):
        c = lax.axis_index("c")
        s = lax.axis_index("s")
        base = s * SLICE
        _zero_acc(acc_sp, vw, base)
        _fill(vw, CHUNK, 1.0)
        plsc.subcore_barrier()
        _edge_pass(None, dst_hbm, None, None, (dv0, dv1, dv2), vw,
                   acc_sp, (si0, si1, si2), (ss0, ss1, ss2),
                   s * NC + c, NW, nchunks, gather=False)
        plsc.subcore_barrier()
        pltpu.sync_copy(acc_sp.at[pl.ds(base, SLICE)],
                        out_hbm.at[c, pl.ds(base, SLICE)])

    return pl.kernel(
        body,
        out_type=jax.ShapeDtypeStruct((NC, NP), jnp.float32),
        mesh=_mesh(),
        compiler_params=_SC_PARAMS,
        scratch_types=[
            pltpu.VMEM_SHARED((NP,), jnp.float32),
            pltpu.VMEM((CHUNK,), jnp.int32),
            pltpu.VMEM((CHUNK,), jnp.int32),
            pltpu.VMEM((CHUNK,), jnp.int32),
            pltpu.VMEM((CHUNK,), jnp.float32),
        ] + [pltpu.SemaphoreType.DMA] * 6,
    )


_SEG_SCRATCH = (
    [pltpu.VMEM_SHARED((NP,), jnp.float32),
     pltpu.VMEM((NP,), jnp.float32)]
    + [pltpu.VMEM((CHUNK,), jnp.int32)] * 6
    + [pltpu.VMEM((CHUNK,), jnp.float32)] * 3
    + [pltpu.SemaphoreType.DMA] * 6
)


@functools.lru_cache(maxsize=None)
def _sc_segsum1(nchunks):
    def body(src_hbm, dst_hbm, g_hbm, out_hbm, acc_sp, tab,
             sv0, sv1, sv2, dv0, dv1, dv2, vw0, vw1, vw2,
             si0, si1, si2, ss0, ss1, ss2):
        c = lax.axis_index("c")
        s = lax.axis_index("s")
        base = s * SLICE
        _zero_acc(acc_sp, vw0, base)
        pltpu.sync_copy(g_hbm, tab)
        plsc.subcore_barrier()
        _edge_pass(src_hbm, dst_hbm, tab, (sv0, sv1, sv2),
                   (dv0, dv1, dv2), (vw0, vw1, vw2), acc_sp,
                   (si0, si1, si2), (ss0, ss1, ss2),
                   s * NC + c, NW, nchunks, gather=True)
        plsc.subcore_barrier()
        pltpu.sync_copy(acc_sp.at[pl.ds(base, SLICE)],
                        out_hbm.at[c, pl.ds(base, SLICE)])

    return pl.kernel(
        body,
        out_type=jax.ShapeDtypeStruct((NC, NP), jnp.float32),
        mesh=_mesh(),
        compiler_params=_SC_PARAMS,
        scratch_types=list(_SEG_SCRATCH),
    )


@functools.lru_cache(maxsize=None)
def _sc_segsum2(nchunks):
    def body(src_hbm, dst_hbm, ga_hbm, gb_hbm, out_hbm, acc_sp, tab,
             sv0, sv1, sv2, dv0, dv1, dv2, vw0, vw1, vw2,
             si0, si1, si2, ss0, ss1, ss2):
        c = lax.axis_index("c")
        s = lax.axis_index("s")
        base = s * SLICE
        _zero_acc(acc_sp, vw0, base)

        @pl.when(c == 0)
        def _():
            pltpu.sync_copy(ga_hbm, tab)

        @pl.when(c == 1)
        def _():
            pltpu.sync_copy(gb_hbm, tab)

        plsc.subcore_barrier()
        _edge_pass(src_hbm, dst_hbm, tab, (sv0, sv1, sv2),
                   (dv0, dv1, dv2), (vw0, vw1, vw2), acc_sp,
                   (si0, si1, si2), (ss0, ss1, ss2),
                   s, NS, nchunks, gather=True)
        plsc.subcore_barrier()
        pltpu.sync_copy(acc_sp.at[pl.ds(base, SLICE)],
                        out_hbm.at[c, pl.ds(base, SLICE)])

    return pl.kernel(
        body,
        out_type=jax.ShapeDtypeStruct((NC, NP), jnp.float32),
        mesh=_mesh(),
        compiler_params=_SC_PARAMS,
        scratch_types=list(_SEG_SCRATCH),
    )


def _tc_prep_body(degp_ref, x_ref, dinv_ref, g1_ref):
    deg = degp_ref[0] + degp_ref[1] + 1.0
    dinv = lax.rsqrt(deg)
    dinv_ref[...] = dinv
    g1_ref[...] = dinv * x_ref[...]


_tc_prep = pl.pallas_call(
    _tc_prep_body,
    out_shape=(jax.ShapeDtypeStruct((NPR, 128), jnp.float32),
               jax.ShapeDtypeStruct((NPR, 128), jnp.float32)),
)


def _tc_mid_body(s_ref, dinv_ref, x_ref, w1_ref, b1_ref, w2_ref,
                 ha_ref, hb_ref, ga_ref, gb_ref):
    dinv = dinv_ref[...]
    t = dinv * (s_ref[0] + s_ref[1]) + dinv * dinv * x_ref[...]
    acc0 = jnp.zeros_like(t)
    acc1 = jnp.zeros_like(t)
    for k in range(16):
        h = jnp.maximum(t * w1_ref[0, k] + b1_ref[0, k], 0.0)
        acc0 = acc0 + h * w2_ref[k, 0]
        acc1 = acc1 + h * w2_ref[k, 1]
    ha_ref[...] = acc0
    hb_ref[...] = acc1
    ga_ref[...] = dinv * acc0
    gb_ref[...] = dinv * acc1


_tc_mid = pl.pallas_call(
    _tc_mid_body,
    out_shape=tuple(jax.ShapeDtypeStruct((NPR, 128), jnp.float32)
                    for _ in range(4)),
)


def _tc_final_body(o_ref, dinv_ref, ha_ref, hb_ref, b2_ref,
                   outa_ref, outb_ref):
    dinv = dinv_ref[...]
    d2 = dinv * dinv
    outa_ref[...] = dinv * o_ref[0] + d2 * ha_ref[...] + b2_ref[0, 0]
    outb_ref[...] = dinv * o_ref[1] + d2 * hb_ref[...] + b2_ref[0, 1]


_tc_final = pl.pallas_call(
    _tc_final_body,
    out_shape=(jax.ShapeDtypeStruct((NPR, 128), jnp.float32),
               jax.ShapeDtypeStruct((NPR, 128), jnp.float32)),
)


def kernel(x, edge_index, W1, b1, W2, b2):
    n = x.shape[0]
    e = edge_index.shape[1]
    if e % CHUNK:
        pad = CHUNK - e % CHUNK
        edge_index = jnp.concatenate(
            [edge_index, jnp.full((2, pad), n, edge_index.dtype)], axis=1)
        e += pad
    nchunks = e // CHUNK
    src = edge_index[0]
    dst = edge_index[1]
    xf = jnp.concatenate(
        [x[:, 0], jnp.zeros((NP - n,), x.dtype)]).reshape(NPR, 128)

    deg2 = _sc_degree(nchunks)(dst).reshape(NC, NPR, 128)
    dinv, g1 = _tc_prep(deg2, xf)
    s2 = _sc_segsum1(nchunks)(src, dst, g1.reshape(NP)).reshape(NC, NPR, 128)
    ha, hb, ga, gb = _tc_mid(s2, dinv, xf, W1, b1.reshape(1, 16), W2)
    o2 = _sc_segsum2(nchunks)(src, dst, ga.reshape(NP), gb.reshape(NP))
    outa, outb = _tc_final(o2.reshape(NC, NPR, 128),
                           dinv, ha, hb, b2.reshape(1, 2))
    return jnp.stack([outa.reshape(NP)[:n], outb.reshape(NP)[:n]], axis=1)
